# Initial kernel scaffold; baseline (speedup 1.0000x reference)
import jax
import jax.numpy as jnp
from jax.experimental import pallas as pl


def kernel(head, tail, kg_src, kg_dst, kg_rel, ent_kg, rel_kg, linear_W, linear_b, relW1, relb1, attW, attb, WrW, Wrb):
    raise NotImplementedError("write your pallas kernel here")


# speedup vs baseline: 2.0207x; 2.0207x over previous
import functools

import jax
import jax.numpy as jnp
from jax import lax
from jax.experimental import pallas as pl
from jax.experimental.pallas import tpu as pltpu
from jax.experimental.pallas import tpu_sc as plsc

N_ENT = 10000
N_DIM = 64
B = 4
N_RELS = 115
N_EDGES = 160000
N_LAYERS = 2
WHALF = (B * N_DIM) // 2

NSUB = 16
K = 128
CHUNKS = 80
PER_SUB = K * CHUNKS
E_PAD = NSUB * PER_SUB
ACC_ROWS = 10240
ROWS_PER_SUB = ACC_ROWS // NSUB


def _rspmm_body(src_hbm, rel_hbm, dst_hbm, h0, h1, r0, r1, zeros_hbm,
                out0, out1, srcv, relv, dstv, hrows, rrows, acc, sem1, sem2):
    c = lax.axis_index("c")
    s = lax.axis_index("s")

    def do_half(h_t, r_t, out_t):
        @pl.when(s == 0)
        def _():
            pltpu.sync_copy(zeros_hbm, acc)

        plsc.subcore_barrier()

        def chunk(i, carry):
            base = s * PER_SUB + i * K
            pltpu.sync_copy(src_hbm.at[pl.ds(base, K)], srcv)
            pltpu.sync_copy(rel_hbm.at[pl.ds(base, K)], relv)
            pltpu.sync_copy(dst_hbm.at[pl.ds(base, K)], dstv)
            g1 = pltpu.async_copy(h_t.at[srcv], hrows, sem1)
            g2 = pltpu.async_copy(r_t.at[relv], rrows, sem2)
            g1.wait()
            g2.wait()

            def mul(j, c2):
                for g in range(WHALF // 16):
                    sl = pl.ds(g * 16, 16)
                    hrows[j, sl] = hrows[j, sl] * rrows[j, sl]
                return c2

            lax.fori_loop(0, K, mul, 0)
            pltpu.sync_copy(hrows, acc.at[dstv], add=True)
            return carry

        lax.fori_loop(0, CHUNKS, chunk, 0)
        plsc.subcore_barrier()
        pltpu.sync_copy(acc.at[pl.ds(s * ROWS_PER_SUB, ROWS_PER_SUB)],
                        out_t.at[pl.ds(s * ROWS_PER_SUB, ROWS_PER_SUB)])

    @pl.when(c == 0)
    def _():
        do_half(h0, r0, out0)

    @pl.when(c == 1)
    def _():
        do_half(h1, r1, out1)


_rspmm = functools.partial(
    pl.kernel,
    mesh=plsc.VectorSubcoreMesh(core_axis_name="c", subcore_axis_name="s"),
    out_type=[
        jax.ShapeDtypeStruct((ACC_ROWS, WHALF), jnp.float32),
        jax.ShapeDtypeStruct((ACC_ROWS, WHALF), jnp.float32),
    ],
    scratch_types=[
        pltpu.VMEM((K,), jnp.int32),
        pltpu.VMEM((K,), jnp.int32),
        pltpu.VMEM((K,), jnp.int32),
        pltpu.VMEM((K, WHALF), jnp.float32),
        pltpu.VMEM((K, WHALF), jnp.float32),
        pltpu.VMEM_SHARED((ACC_ROWS, WHALF), jnp.float32),
        pltpu.SemaphoreType.DMA,
        pltpu.SemaphoreType.DMA,
    ],
)(_rspmm_body)


def _prep_body(ht_ref, rw1_ref, rb1_ref, aw_ref, ab_ref, rk_ref, ro_ref):
    ht = ht_ref[:, :]
    for l in range(N_LAYERS):
        t = jnp.maximum(
            jnp.dot(ht, rw1_ref[l], preferred_element_type=jnp.float32)
            + rb1_ref[l], 0.0)
        t = jax.nn.sigmoid(
            jnp.dot(t, aw_ref[l], preferred_element_type=jnp.float32)
            + ab_ref[l])
        ro_ref[l] = t.T[:, :, None] * rk_ref[l][:, None, :]


def _prep(ht, relW1, relb1, attW, attb, rel_kg):
    return pl.pallas_call(
        _prep_body,
        out_shape=jax.ShapeDtypeStruct((N_LAYERS, N_RELS, B, N_DIM),
                                       jnp.float32),
    )(ht, relW1, relb1, attW, attb, rel_kg)


def _linear_body(x_ref, w_ref, b_ref, o_ref):
    o_ref[:, :] = jnp.maximum(
        jnp.dot(x_ref[:, :], w_ref[:, :], preferred_element_type=jnp.float32)
        + b_ref[0, :], 0.0)


def _linear_relu(x, w, b):
    rows = x.shape[0]
    blk = 2000
    grid = rows // blk
    return pl.pallas_call(
        _linear_body,
        grid=(grid,),
        in_specs=[
            pl.BlockSpec((blk, N_DIM), lambda i: (i, 0)),
            pl.BlockSpec((N_DIM, N_DIM), lambda i: (0, 0)),
            pl.BlockSpec((1, N_DIM), lambda i: (0, 0)),
        ],
        out_specs=pl.BlockSpec((blk, N_DIM), lambda i: (i, 0)),
        out_shape=jax.ShapeDtypeStruct((rows, N_DIM), jnp.float32),
    )(x, w, b.reshape(1, N_DIM))


def _score_body(e_ref, w_ref, b_ref, o_ref):
    o_ref[:, :] = (
        jnp.dot(e_ref[:, :], w_ref[:, :], preferred_element_type=jnp.float32)
        + b_ref[0, :])


def _scores(emb, WrW, Wrb):
    return pl.pallas_call(
        _score_body,
        out_shape=jax.ShapeDtypeStruct((B, WrW.shape[1]), jnp.float32),
    )(emb, WrW, Wrb.reshape(1, -1))


def kernel(head, tail, kg_src, kg_dst, kg_rel, ent_kg, rel_kg, linear_W,
           linear_b, relW1, relb1, attW, attb, WrW, Wrb):
    head = head.astype(jnp.int32)
    tail = tail.astype(jnp.int32)
    src = kg_src.astype(jnp.int32)
    dst = kg_dst.astype(jnp.int32)
    rel = kg_rel.astype(jnp.int32)

    head_embed = ent_kg[head]
    tail_embed = ent_kg[tail]
    ht = jnp.concatenate([head_embed, tail_embed], axis=-1)

    rel_out = _prep(ht, relW1, relb1, attW, attb, rel_kg)
    rel_flat = rel_out.reshape(N_LAYERS, N_RELS, B * N_DIM)
    r_half = [(rel_flat[l, :, :WHALF], rel_flat[l, :, WHALF:])
              for l in range(N_LAYERS)]

    pad = E_PAD - N_EDGES
    src_p = jnp.pad(src, (0, pad))
    rel_p = jnp.pad(rel, (0, pad))
    dst_p = jnp.pad(dst, (0, pad), constant_values=N_ENT)
    zeros = jnp.zeros((ACC_ROWS, WHALF), jnp.float32)

    def propagate(init_embed, init_idx):
        h = jnp.zeros((N_ENT, B, N_DIM), jnp.float32)
        h = h.at[init_idx, jnp.arange(B)].set(init_embed)
        h = h.reshape(N_ENT, B * N_DIM)
        for l in range(N_LAYERS):
            h3 = h.reshape(N_ENT, 2, WHALF)
            o0, o1 = _rspmm(src_p, rel_p, dst_p,
                            h3[:, 0], h3[:, 1],
                            r_half[l][0], r_half[l][1], zeros)
            msg = jnp.stack([o0[:N_ENT], o1[:N_ENT]], axis=1)
            msg = msg.reshape(N_ENT * B, N_DIM)
            h = _linear_relu(msg, linear_W[l], linear_b[l])
            h = h.reshape(N_ENT, B * N_DIM)
        return h.reshape(N_ENT, B, N_DIM)

    tail_hid = propagate(head_embed, head)[tail, jnp.arange(B)]
    head_hid = propagate(tail_embed, tail)[head, jnp.arange(B)]
    emb = jnp.concatenate([head_embed, head_hid, tail_hid, tail_embed],
                          axis=1)
    return _scores(emb, WrW, Wrb)


# speedup vs baseline: 2.4780x; 1.2263x over previous
import functools

import jax
import jax.numpy as jnp
from jax import lax
from jax.experimental import pallas as pl
from jax.experimental.pallas import tpu as pltpu
from jax.experimental.pallas import tpu_sc as plsc

N_ENT = 10000
N_DIM = 64
B = 4
N_RELS = 115
N_EDGES = 160000
N_LAYERS = 2
WHALF = (B * N_DIM) // 2

NSUB = 16
K = 64
CHUNKS = 160
PER_SUB = K * CHUNKS
E_PAD = NSUB * PER_SUB
ACC_ROWS = 10240
ROWS_PER_SUB = ACC_ROWS // NSUB


def _rspmm_body(src_hbm, rel_hbm, dst_hbm, h0, h1, r0, r1, zeros_hbm,
                out0, out1, srcv, relv, dstv, hrows0, rrows0, hrows1,
                rrows1, acc, sh0, sr0, sh1, sr1):
    c = lax.axis_index("c")
    s = lax.axis_index("s")
    bufs = ((hrows0, rrows0, sh0, sr0), (hrows1, rrows1, sh1, sr1))

    def do_half(h_t, r_t, out_t):
        @pl.when(s == 0)
        def _():
            pltpu.sync_copy(zeros_hbm, acc)

        plsc.subcore_barrier()

        def load_idx(i, bi):
            base = s * PER_SUB + i * K
            pltpu.sync_copy(src_hbm.at[pl.ds(base, K)], srcv.at[bi])
            pltpu.sync_copy(rel_hbm.at[pl.ds(base, K)], relv.at[bi])
            pltpu.sync_copy(dst_hbm.at[pl.ds(base, K)], dstv.at[bi])

        def start(bi):
            h_b, r_b, sh, sr = bufs[bi]
            pltpu.async_copy(h_t.at[srcv.at[bi]], h_b, sh)
            pltpu.async_copy(r_t.at[relv.at[bi]], r_b, sr)

        def finish(i, bi):
            h_b, r_b, sh, sr = bufs[bi]
            pltpu.make_async_copy(h_t.at[pl.ds(0, K)], h_b, sh).wait()
            pltpu.make_async_copy(r_t.at[pl.ds(0, K)], r_b, sr).wait()

            def mul(j, c2):
                for g in range(WHALF // 16):
                    sl = pl.ds(g * 16, 16)
                    h_b[j, sl] = h_b[j, sl] * r_b[j, sl]
                return c2

            lax.fori_loop(0, K, mul, 0)
            pltpu.sync_copy(h_b, acc.at[dstv.at[bi]], add=True)

        load_idx(0, 0)
        start(0)

        def pair(t, carry):
            for u in range(2):
                i = 2 * t + u

                @pl.when(i + 1 < CHUNKS)
                def _():
                    load_idx(i + 1, 1 - u)
                    start(1 - u)

                finish(i, u)
            return carry

        lax.fori_loop(0, CHUNKS // 2, pair, 0)
        plsc.subcore_barrier()
        pltpu.sync_copy(acc.at[pl.ds(s * ROWS_PER_SUB, ROWS_PER_SUB)],
                        out_t.at[pl.ds(s * ROWS_PER_SUB, ROWS_PER_SUB)])

    @pl.when(c == 0)
    def _():
        do_half(h0, r0, out0)

    @pl.when(c == 1)
    def _():
        do_half(h1, r1, out1)


_rspmm = functools.partial(
    pl.kernel,
    mesh=plsc.VectorSubcoreMesh(core_axis_name="c", subcore_axis_name="s"),
    out_type=[
        jax.ShapeDtypeStruct((ACC_ROWS, WHALF), jnp.float32),
        jax.ShapeDtypeStruct((ACC_ROWS, WHALF), jnp.float32),
    ],
    scratch_types=[
        pltpu.VMEM((2, K), jnp.int32),
        pltpu.VMEM((2, K), jnp.int32),
        pltpu.VMEM((2, K), jnp.int32),
        pltpu.VMEM((K, WHALF), jnp.float32),
        pltpu.VMEM((K, WHALF), jnp.float32),
        pltpu.VMEM((K, WHALF), jnp.float32),
        pltpu.VMEM((K, WHALF), jnp.float32),
        pltpu.VMEM_SHARED((ACC_ROWS, WHALF), jnp.float32),
        pltpu.SemaphoreType.DMA,
        pltpu.SemaphoreType.DMA,
        pltpu.SemaphoreType.DMA,
        pltpu.SemaphoreType.DMA,
    ],
)(_rspmm_body)


def _prep_body(ht_ref, rw1_ref, rb1_ref, aw_ref, ab_ref, rk_ref, ro_ref):
    ht = ht_ref[:, :]
    for l in range(N_LAYERS):
        t = jnp.maximum(
            jnp.dot(ht, rw1_ref[l], preferred_element_type=jnp.float32)
            + rb1_ref[l], 0.0)
        t = jax.nn.sigmoid(
            jnp.dot(t, aw_ref[l], preferred_element_type=jnp.float32)
            + ab_ref[l])
        ro_ref[l] = t.T[:, :, None] * rk_ref[l][:, None, :]


def _prep(ht, relW1, relb1, attW, attb, rel_kg):
    return pl.pallas_call(
        _prep_body,
        out_shape=jax.ShapeDtypeStruct((N_LAYERS, N_RELS, B, N_DIM),
                                       jnp.float32),
    )(ht, relW1, relb1, attW, attb, rel_kg)


def _linear_body(x_ref, w_ref, b_ref, o_ref):
    o_ref[:, :] = jnp.maximum(
        jnp.dot(x_ref[:, :], w_ref[:, :], preferred_element_type=jnp.float32)
        + b_ref[0, :], 0.0)


def _linear_relu(x, w, b):
    rows = x.shape[0]
    blk = 2000
    grid = rows // blk
    return pl.pallas_call(
        _linear_body,
        grid=(grid,),
        in_specs=[
            pl.BlockSpec((blk, N_DIM), lambda i: (i, 0)),
            pl.BlockSpec((N_DIM, N_DIM), lambda i: (0, 0)),
            pl.BlockSpec((1, N_DIM), lambda i: (0, 0)),
        ],
        out_specs=pl.BlockSpec((blk, N_DIM), lambda i: (i, 0)),
        out_shape=jax.ShapeDtypeStruct((rows, N_DIM), jnp.float32),
    )(x, w, b.reshape(1, N_DIM))


def _score_body(e_ref, w_ref, b_ref, o_ref):
    o_ref[:, :] = (
        jnp.dot(e_ref[:, :], w_ref[:, :], preferred_element_type=jnp.float32)
        + b_ref[0, :])


def _scores(emb, WrW, Wrb):
    return pl.pallas_call(
        _score_body,
        out_shape=jax.ShapeDtypeStruct((B, WrW.shape[1]), jnp.float32),
    )(emb, WrW, Wrb.reshape(1, -1))


def kernel(head, tail, kg_src, kg_dst, kg_rel, ent_kg, rel_kg, linear_W,
           linear_b, relW1, relb1, attW, attb, WrW, Wrb):
    head = head.astype(jnp.int32)
    tail = tail.astype(jnp.int32)
    src = kg_src.astype(jnp.int32)
    dst = kg_dst.astype(jnp.int32)
    rel = kg_rel.astype(jnp.int32)

    head_embed = ent_kg[head]
    tail_embed = ent_kg[tail]
    ht = jnp.concatenate([head_embed, tail_embed], axis=-1)

    rel_out = _prep(ht, relW1, relb1, attW, attb, rel_kg)
    rel_flat = rel_out.reshape(N_LAYERS, N_RELS, B * N_DIM)
    r_half = [(rel_flat[l, :, :WHALF], rel_flat[l, :, WHALF:])
              for l in range(N_LAYERS)]

    pad = E_PAD - N_EDGES
    src_p = jnp.pad(src, (0, pad))
    rel_p = jnp.pad(rel, (0, pad))
    dst_p = jnp.pad(dst, (0, pad), constant_values=N_ENT)
    zeros = jnp.zeros((ACC_ROWS, WHALF), jnp.float32)

    def propagate(init_embed, init_idx):
        h = jnp.zeros((N_ENT, B, N_DIM), jnp.float32)
        h = h.at[init_idx, jnp.arange(B)].set(init_embed)
        h = h.reshape(N_ENT, B * N_DIM)
        for l in range(N_LAYERS):
            h3 = h.reshape(N_ENT, 2, WHALF)
            o0, o1 = _rspmm(src_p, rel_p, dst_p,
                            h3[:, 0], h3[:, 1],
                            r_half[l][0], r_half[l][1], zeros)
            msg = jnp.stack([o0[:N_ENT], o1[:N_ENT]], axis=1)
            msg = msg.reshape(N_ENT * B, N_DIM)
            h = _linear_relu(msg, linear_W[l], linear_b[l])
            h = h.reshape(N_ENT, B * N_DIM)
        return h.reshape(N_ENT, B, N_DIM)

    tail_hid = propagate(head_embed, head)[tail, jnp.arange(B)]
    head_hid = propagate(tail_embed, tail)[head, jnp.arange(B)]
    emb = jnp.concatenate([head_embed, head_hid, tail_hid, tail_embed],
                          axis=1)
    return _scores(emb, WrW, Wrb)
